# baseline (speedup 1.0000x reference)
import jax
import jax.numpy as jnp
from jax import lax
from jax.experimental import pallas as pl

NUM_CLUSTERS = 64
TOPK = 5
KMEANS_ITERS = 3
HEADS = 12
DIM_HEAD = 64

_NEG_NONMEMBER = -1e9
_NEG_INVALID = -2e9
_NEG_PICKED = -3e9

_F32 = jnp.float32
_BF16 = jnp.bfloat16


def _dot(a, b, precision=None):
    return lax.dot_general(a, b, ((((1,), (0,))), ((), ())),
                           preferred_element_type=_F32, precision=precision)


def _dotx(a, b):
    return _dot(a, b, precision=lax.Precision.HIGHEST)


def _cluster_select_body(x_ref, wk_ref, wv_ref, kt_ref, v_ref):
    X = x_ref[0]
    N, D = X.shape
    C = NUM_CLUSTERS
    XT = jnp.transpose(X)
    Xbf = X.astype(_BF16)
    XTbf = XT.astype(_BF16)

    tT = lax.broadcasted_iota(jnp.int32, (C, N), 1)
    cT = lax.broadcasted_iota(jnp.int32, (C, N), 0)
    validT = tT >= 1

    sel0 = (tT == cT + 1).astype(_F32)
    cen = _dotx(sel0, X)

    def kmeans_iter(_, carry):
        cen, _ = carry
        cnorm = jnp.sum(cen * cen, axis=1, keepdims=True)
        AT = _dot(cen.astype(_BF16), XTbf)
        scores = cnorm - 2.0 * AT
        minv = jnp.min(scores, axis=0, keepdims=True)
        assign = jnp.min(jnp.where(scores == minv, cT, C), axis=0,
                         keepdims=True)
        onehot = ((cT == assign) & validT).astype(_F32)
        counts = jnp.maximum(jnp.sum(onehot, axis=1, keepdims=True), 1.0)
        return _dot(onehot.astype(_BF16), Xbf) / counts, onehot

    cen, onehot = lax.fori_loop(
        0, KMEANS_ITERS, kmeans_iter,
        (cen, jnp.zeros((C, N), _F32)))

    sims = _dot(cen.astype(_BF16), XTbf)
    masked = jnp.where(onehot > 0.5, sims,
                       jnp.where(validT, _NEG_NONMEMBER, _NEG_INVALID))

    Wk = wk_ref[...].astype(_BF16)
    Wv = wv_ref[...].astype(_BF16)
    for j in range(TOPK):
        mx = jnp.max(masked, axis=1, keepdims=True)
        selcol = jnp.min(jnp.where(masked == mx, tT, N), axis=1,
                         keepdims=True)
        selmask = tT == selcol
        xk_j = _dot(selmask.astype(_BF16), Xbf).astype(_BF16)
        k_j = _dot(xk_j, Wk).astype(_BF16)
        kt_ref[0, :, j * C:(j + 1) * C] = jnp.transpose(k_j)
        v_ref[0, j * C:(j + 1) * C, :] = _dot(xk_j, Wv).astype(_BF16)
        masked = jnp.where(selmask, _NEG_PICKED, masked)


def _attn_body(x_ref, kt_ref, v_ref, wq_ref, wo_ref, b_ref, out_ref):
    DH = DIM_HEAD
    Xb = x_ref[0].astype(_BF16)
    q = _dot(Xb, wq_ref[...].astype(_BF16)).astype(_BF16)
    scale = DH ** -0.5
    ys = []
    for h in range(HEADS):
        kt_h = kt_ref[0, h * DH:(h + 1) * DH, :]
        v_h = v_ref[0][:, h * DH:(h + 1) * DH]
        dots = _dot(q[:, h * DH:(h + 1) * DH], kt_h) * scale
        m = jnp.max(dots, axis=1, keepdims=True)
        e = jnp.exp(dots - m)
        s_inv = 1.0 / jnp.sum(e, axis=1, keepdims=True)
        y = _dot(e.astype(_BF16), v_h) * s_inv
        ys.append(y.astype(_BF16))
    Yall = jnp.concatenate(ys, axis=1)
    out_ref[0] = _dot(Yall, wo_ref[...].astype(_BF16)) + b_ref[0]

    @pl.when(pl.program_id(1) == 0)
    def _():
        out_ref[0, 0:1, :] = x_ref[0, 0:1, :]


@jax.jit
def kernel(x, W_qkv, W_out, b_out):
    B, N, D = x.shape
    C = NUM_CLUSTERS
    M = C * TOPK
    H, DH = HEADS, DIM_HEAD
    inner = H * DH

    b2 = b_out.reshape(1, D)

    kt, vh = pl.pallas_call(
        _cluster_select_body,
        grid=(B,),
        in_specs=[
            pl.BlockSpec((1, N, D), lambda b: (b, 0, 0)),
            pl.BlockSpec((D, inner), lambda b: (0, 1)),
            pl.BlockSpec((D, inner), lambda b: (0, 2)),
        ],
        out_specs=[
            pl.BlockSpec((1, inner, M), lambda b: (b, 0, 0)),
            pl.BlockSpec((1, M, inner), lambda b: (b, 0, 0)),
        ],
        out_shape=[
            jax.ShapeDtypeStruct((B, inner, M), jnp.bfloat16),
            jax.ShapeDtypeStruct((B, M, inner), jnp.bfloat16),
        ],
    )(x, W_qkv, W_qkv)

    NB = 1024
    n_blocks = N // NB
    Y = pl.pallas_call(
        _attn_body,
        grid=(B, n_blocks),
        in_specs=[
            pl.BlockSpec((1, NB, D), lambda b, n: (b, n, 0)),
            pl.BlockSpec((1, inner, M), lambda b, n: (b, 0, 0)),
            pl.BlockSpec((1, M, inner), lambda b, n: (b, 0, 0)),
            pl.BlockSpec((D, inner), lambda b, n: (0, 0)),
            pl.BlockSpec((inner, D), lambda b, n: (0, 0)),
            pl.BlockSpec((1, D), lambda b, n: (0, 0)),
        ],
        out_specs=pl.BlockSpec((1, NB, D), lambda b, n: (b, n, 0)),
        out_shape=jax.ShapeDtypeStruct((B, N, D), _F32),
    )(x, kt, vh, W_qkv, W_out, b2)

    return Y


# speedup vs baseline: 1.1045x; 1.1045x over previous
import jax
import jax.numpy as jnp
from jax import lax
from jax.experimental import pallas as pl

NUM_CLUSTERS = 64
TOPK = 5
KMEANS_ITERS = 3
HEADS = 12
DIM_HEAD = 64

_NEG_NONMEMBER = -1e9
_NEG_INVALID = -2e9
_NEG_PICKED = -3e9

_F32 = jnp.float32
_BF16 = jnp.bfloat16


def _dot(a, b, precision=None):
    return lax.dot_general(a, b, ((((1,), (0,))), ((), ())),
                           preferred_element_type=_F32, precision=precision)


def _dotx(a, b):
    return _dot(a, b, precision=lax.Precision.HIGHEST)


def _cluster_select_body(x_ref, wk_ref, wv_ref, kt_ref, v_ref):
    X = x_ref[0]
    N, D = X.shape
    C = NUM_CLUSTERS
    XT = jnp.transpose(X)
    Xbf = X.astype(_BF16)
    XTbf = XT.astype(_BF16)

    tT = lax.broadcasted_iota(jnp.int32, (C, N), 1)
    cT = lax.broadcasted_iota(jnp.int32, (C, N), 0)
    validT = tT >= 1

    sel0 = (tT == cT + 1).astype(_F32)
    cen = _dotx(sel0, X)

    def kmeans_iter(_, carry):
        cen, _ = carry
        cnorm = jnp.sum(cen * cen, axis=1, keepdims=True)
        AT = _dot(cen.astype(_BF16), XTbf)
        scores = cnorm - 2.0 * AT
        minv = jnp.min(scores, axis=0, keepdims=True)
        assign = jnp.min(jnp.where(scores == minv, cT, C), axis=0,
                         keepdims=True)
        onehot = ((cT == assign) & validT).astype(_F32)
        counts = jnp.maximum(jnp.sum(onehot, axis=1, keepdims=True), 1.0)
        return _dot(onehot.astype(_BF16), Xbf) / counts, onehot

    cen, onehot = lax.fori_loop(
        0, KMEANS_ITERS, kmeans_iter,
        (cen, jnp.zeros((C, N), _F32)))

    sims = _dot(cen.astype(_BF16), XTbf)
    masked = jnp.where(onehot > 0.5, sims,
                       jnp.where(validT, _NEG_NONMEMBER, _NEG_INVALID))

    Wk = wk_ref[...].astype(_BF16)
    Wv = wv_ref[...].astype(_BF16)
    for j in range(TOPK):
        mx = jnp.max(masked, axis=1, keepdims=True)
        selcol = jnp.min(jnp.where(masked == mx, tT, N), axis=1,
                         keepdims=True)
        selmask = tT == selcol
        xk_j = _dot(selmask.astype(_BF16), Xbf).astype(_BF16)
        k_j = (_dot(xk_j, Wk) * (DIM_HEAD ** -0.5)).astype(_BF16)
        kt_ref[0, :, j * C:(j + 1) * C] = jnp.transpose(k_j)
        v_ref[0, j * C:(j + 1) * C, :] = _dot(xk_j, Wv).astype(_BF16)
        masked = jnp.where(selmask, _NEG_PICKED, masked)


def _attn_body(x_ref, kt_ref, v_ref, wq_ref, wo_ref, b_ref, out_ref):
    DH = DIM_HEAD
    Xb = x_ref[0].astype(_BF16)
    q = _dot(Xb, wq_ref[...].astype(_BF16)).astype(_BF16)
    ys = []
    for h in range(HEADS):
        kt_h = kt_ref[0, h * DH:(h + 1) * DH, :]
        v_h = v_ref[0][:, h * DH:(h + 1) * DH]
        dots = _dot(q[:, h * DH:(h + 1) * DH], kt_h)
        m = jnp.max(dots, axis=1, keepdims=True)
        e = jnp.exp(dots - m)
        p = e / jnp.sum(e, axis=1, keepdims=True)
        ys.append(_dot(p.astype(_BF16), v_h).astype(_BF16))
    Yall = jnp.concatenate(ys, axis=1)
    out_ref[0] = _dot(Yall, wo_ref[...].astype(_BF16)) + b_ref[0]

    @pl.when(pl.program_id(1) == 0)
    def _():
        out_ref[0, 0:1, :] = x_ref[0, 0:1, :]


@jax.jit
def kernel(x, W_qkv, W_out, b_out):
    B, N, D = x.shape
    C = NUM_CLUSTERS
    M = C * TOPK
    H, DH = HEADS, DIM_HEAD
    inner = H * DH

    b2 = b_out.reshape(1, D)

    kt, vh = pl.pallas_call(
        _cluster_select_body,
        grid=(B,),
        in_specs=[
            pl.BlockSpec((1, N, D), lambda b: (b, 0, 0)),
            pl.BlockSpec((D, inner), lambda b: (0, 1)),
            pl.BlockSpec((D, inner), lambda b: (0, 2)),
        ],
        out_specs=[
            pl.BlockSpec((1, inner, M), lambda b: (b, 0, 0)),
            pl.BlockSpec((1, M, inner), lambda b: (b, 0, 0)),
        ],
        out_shape=[
            jax.ShapeDtypeStruct((B, inner, M), jnp.bfloat16),
            jax.ShapeDtypeStruct((B, M, inner), jnp.bfloat16),
        ],
    )(x, W_qkv, W_qkv)

    NB = 2048
    n_blocks = N // NB
    Y = pl.pallas_call(
        _attn_body,
        grid=(B, n_blocks),
        in_specs=[
            pl.BlockSpec((1, NB, D), lambda b, n: (b, n, 0)),
            pl.BlockSpec((1, inner, M), lambda b, n: (b, 0, 0)),
            pl.BlockSpec((1, M, inner), lambda b, n: (b, 0, 0)),
            pl.BlockSpec((D, inner), lambda b, n: (0, 0)),
            pl.BlockSpec((inner, D), lambda b, n: (0, 0)),
            pl.BlockSpec((1, D), lambda b, n: (0, 0)),
        ],
        out_specs=pl.BlockSpec((1, NB, D), lambda b, n: (b, n, 0)),
        out_shape=jax.ShapeDtypeStruct((B, N, D), _F32),
    )(x, kt, vh, W_qkv, W_out, b2)

    return Y
